# Initial kernel scaffold; baseline (speedup 1.0000x reference)
import jax
import jax.numpy as jnp
from jax.experimental import pallas as pl


def kernel(s, prog, lib_W, emb_table):
    raise NotImplementedError("write your pallas kernel here")


# speedup vs baseline: 27.0901x; 27.0901x over previous
import jax
import jax.numpy as jnp
from jax.experimental import pallas as pl
from jax.experimental.pallas import tpu as pltpu

_T = 20
_L = 8
_D = 64
_BT = 2048


def _body(prog_ref, s_ref, wcat_ref, emb_ref, out_ref, tr_ref):
    prog = prog_ref[...]
    sel = jax.nn.softmax(prog, axis=-1)
    wcat = wcat_ref[...]
    bias = jnp.dot(emb_ref[0:_T, :], wcat,
                   preferred_element_type=jnp.float32)
    cur = s_ref[...]
    for i in range(_T):
        y = jnp.tanh(
            jnp.dot(cur, wcat, preferred_element_type=jnp.float32)
            + bias[i:i + 1, :])
        acc = y[:, 0:_D] * sel[i:i + 1, 0:1]
        for l in range(1, _L):
            acc = acc + y[:, l * _D:(l + 1) * _D] * sel[i:i + 1, l:l + 1]
        cur = acc
    out_ref[...] = cur
    tr_ref[...] = prog


def kernel(s, prog, lib_W, emb_table):
    B = s.shape[0]
    wcat = jnp.transpose(lib_W, (1, 0, 2)).reshape(_D, _L * _D)
    grid = (B // _BT,)
    out, trace = pl.pallas_call(
        _body,
        grid=grid,
        in_specs=[
            pl.BlockSpec((_T, _L), lambda t: (0, 0)),
            pl.BlockSpec((_BT, _D), lambda t: (t, 0)),
            pl.BlockSpec((_D, _L * _D), lambda t: (0, 0)),
            pl.BlockSpec((32, _D), lambda t: (0, 0)),
        ],
        out_specs=[
            pl.BlockSpec((_BT, _D), lambda t: (t, 0)),
            pl.BlockSpec((_T, _L), lambda t: (0, 0)),
        ],
        out_shape=[
            jax.ShapeDtypeStruct((B, _D), jnp.float32),
            jax.ShapeDtypeStruct((_T, _L), jnp.float32),
        ],
        compiler_params=pltpu.CompilerParams(
            dimension_semantics=("arbitrary",),
        ),
    )(prog, s, wcat, emb_table)
    return (out, trace)
